# baseline (speedup 1.0000x reference)
import functools

import jax
import jax.numpy as jnp
from jax import lax
from jax.experimental import pallas as pl
from jax.experimental.pallas import tpu as pltpu
from jax.experimental.pallas import tpu_sc as plsc

TOTAL = 16384
D = 2048
SEG = 2048
NSEG = TOTAL // SEG
BM = 1024
M_TILES = SEG // BM
RCHUNK = 16
NITER = SEG // RCHUNK

_INFO = plsc.get_sparse_core_info()
_NC = _INFO.num_cores


def _make_sc_mean():
    mesh = plsc.VectorSubcoreMesh(core_axis_name="c", subcore_axis_name="s")

    @functools.partial(
        pl.kernel, mesh=mesh,
        out_type=jax.ShapeDtypeStruct((NSEG, D), jnp.float32),
        scratch_types=[
            pltpu.VMEM((RCHUNK, D), jnp.float32),
            pltpu.VMEM((D,), jnp.float32),
        ],
    )
    def sc_mean(x_hbm, out_hbm, chunk_v, acc_v):
        wid = lax.axis_index("s") * _NC + lax.axis_index("c")

        @pl.when(wid < NSEG)
        def _():
            acc_v[...] = jnp.zeros((D,), jnp.float32)

            def body(i, carry):
                pltpu.sync_copy(
                    x_hbm.at[pl.ds(wid * SEG + i * RCHUNK, RCHUNK), :],
                    chunk_v)
                for r in range(RCHUNK):
                    acc_v[...] = acc_v[...] + chunk_v[r]
                return carry

            lax.fori_loop(0, NITER, body, 0)
            acc_v[...] = acc_v[...] * (1.0 / SEG)
            pltpu.sync_copy(acc_v, out_hbm.at[wid])

    return sc_mean


_sc_mean = _make_sc_mean()


def _mm_body(x_ref, xm_ref, w_ref, b_ref, o_ref):
    xc = (x_ref[...] - xm_ref[0]).astype(jnp.bfloat16)
    o_ref[...] = jax.lax.dot_general(
        xc, w_ref[...],
        dimension_numbers=(((1,), (1,)), ((), ())),
        preferred_element_type=jnp.float32,
    ) + b_ref[...]


def kernel(x, W, b, l):
    b_eff = (b + (jnp.asarray(l) - SEG).astype(jnp.float32)).reshape(1, D)
    W_bf = W.astype(jnp.bfloat16)

    xm = _sc_mean(x).reshape(NSEG, 1, D)

    out = pl.pallas_call(
        _mm_body,
        grid=(TOTAL // BM,),
        in_specs=[
            pl.BlockSpec((BM, D), lambda i: (i, 0)),
            pl.BlockSpec((1, 1, D), lambda i: (i // M_TILES, 0, 0)),
            pl.BlockSpec((D, D), lambda i: (0, 0)),
            pl.BlockSpec((1, D), lambda i: (0, 0)),
        ],
        out_specs=pl.BlockSpec((BM, D), lambda i: (i, 0)),
        out_shape=jax.ShapeDtypeStruct((TOTAL, D), jnp.float32),
        compiler_params=pltpu.CompilerParams(
            vmem_limit_bytes=64 * 1024 * 1024,
        ),
    )(x, xm, W_bf, b_eff)
    return out


# speedup vs baseline: 8.3064x; 8.3064x over previous
import jax
import jax.numpy as jnp
from jax.experimental import pallas as pl
from jax.experimental.pallas import tpu as pltpu

TOTAL = 16384
D = 2048
SEG = 2048
NSEG = TOTAL // SEG
BM = 1024
M_TILES = SEG // BM


def _fused_body(x_ref, w_ref, b_ref, o_ref, xm_ref):
    m = pl.program_id(1)

    @pl.when(m == 0)
    def _():
        xm_ref[...] = jnp.mean(x_ref[...], axis=0, keepdims=True)

    xc = (x_ref[pl.ds(m * BM, BM), :] - xm_ref[...]).astype(jnp.bfloat16)
    o_ref[...] = jax.lax.dot_general(
        xc, w_ref[...],
        dimension_numbers=(((1,), (1,)), ((), ())),
        preferred_element_type=jnp.float32,
    ) + b_ref[...]


def kernel(x, W, b, l):
    b_eff = (b + (jnp.asarray(l) - SEG).astype(jnp.float32)).reshape(1, D)
    W_bf = W.astype(jnp.bfloat16)

    out = pl.pallas_call(
        _fused_body,
        grid=(NSEG, M_TILES),
        in_specs=[
            pl.BlockSpec((SEG, D), lambda s, m: (s, 0)),
            pl.BlockSpec((D, D), lambda s, m: (0, 0)),
            pl.BlockSpec((1, D), lambda s, m: (0, 0)),
        ],
        out_specs=pl.BlockSpec((BM, D), lambda s, m: (s * M_TILES + m, 0)),
        out_shape=jax.ShapeDtypeStruct((TOTAL, D), jnp.float32),
        scratch_shapes=[pltpu.VMEM((1, D), jnp.float32)],
        compiler_params=pltpu.CompilerParams(
            vmem_limit_bytes=64 * 1024 * 1024,
        ),
    )(x, W_bf, b_eff)
    return out
